# baseline (speedup 1.0000x reference)
import functools

import jax
import jax.numpy as jnp
from jax import lax
from jax.experimental import pallas as pl
from jax.experimental.pallas import tpu as pltpu
from jax.experimental.pallas import tpu_sc as plsc

N = 10000
E = 320000
F_IN = 128
H = 64
C_OUT = 16
LSTM_H = 128

NC = 2
NS = 16
NW = NC * NS

ACC_ROWS = 10240
ROWS_PER_TILE = ACC_ROWS // NS
EDGE_CHUNK = 128
EP = 327680
EDGES_PER_TILE = EP // NW
CHUNKS_PER_TILE = EDGES_PER_TILE // EDGE_CHUNK
DEG_W = 16

BN = 2000
GRID = N // BN

_HI = jax.lax.Precision.DEFAULT


def _fill_buf(buf, nrow, ncolv, value):
    v = jnp.full((16,), value, jnp.float32)

    def body(t, carry):
        i = t // ncolv
        j = t % ncolv
        buf[i, pl.ds(j * 16, 16)] = v
        return carry

    lax.fori_loop(0, nrow * ncolv, body, 0)


NBUF = 5


def _sc_degree(dst2):
    mesh = plsc.VectorSubcoreMesh(core_axis_name="c", subcore_axis_name="s")

    @functools.partial(
        pl.kernel,
        mesh=mesh,
        out_type=jax.ShapeDtypeStruct((NC * ACC_ROWS, DEG_W), jnp.float32),
        compiler_params=pltpu.CompilerParams(use_tc_tiling_on_sc=False),
        scratch_types=[
            pltpu.VMEM_SHARED((ACC_ROWS, DEG_W), jnp.float32),
            pltpu.VMEM((NBUF, EDGE_CHUNK), jnp.int32),
            pltpu.VMEM((EDGE_CHUNK, DEG_W), jnp.float32),
            pltpu.VMEM((EDGE_CHUNK, DEG_W), jnp.float32),
        ]
        + [pltpu.SemaphoreType.DMA] * (2 * NBUF),
    )
    def k(dst_hbm, out_hbm, acc, dstI, ones_b, zero_b, *sems):
        cid = lax.axis_index("c")
        sid = lax.axis_index("s")
        wid = cid * NS + sid
        isems = sems[:NBUF]
        ssems = sems[NBUF:]

        _fill_buf(zero_b, EDGE_CHUNK, DEG_W // 16, 0.0)
        _fill_buf(ones_b, EDGE_CHUNK, DEG_W // 16, 1.0)

        row0 = sid * ROWS_PER_TILE
        for j in range(ROWS_PER_TILE // EDGE_CHUNK):
            pltpu.sync_copy(zero_b, acc.at[pl.ds(row0 + j * EDGE_CHUNK, EDGE_CHUNK)])
        plsc.subcore_barrier()

        crow0 = wid * CHUNKS_PER_TILE

        def fire_idx(kk, b):
            pltpu.async_copy(dst_hbm.at[crow0 + kk], dstI.at[b], isems[b])

        def wait_idx(b):
            pltpu.make_async_copy(dst_hbm.at[0], dstI.at[b], isems[b]).wait()

        def fire_scatter(b):
            pltpu.async_copy(ones_b, acc.at[dstI.at[b]], ssems[b], add=True)

        def wait_scatter(b):
            pltpu.make_async_copy(ones_b, acc.at[dstI.at[b]], ssems[b]).wait()

        fire_idx(0, 0)
        fire_idx(1, 1)

        def group(g, carry):
            for b in range(NBUF):
                kk = g * NBUF + b
                b2 = (b + 2) % NBUF

                @pl.when(kk >= NBUF - 2)
                def _():
                    wait_scatter(b2)

                @pl.when(kk + 2 < CHUNKS_PER_TILE)
                def _():
                    fire_idx(kk + 2, b2)

                wait_idx(b)
                fire_scatter(b)
            return carry

        lax.fori_loop(0, CHUNKS_PER_TILE // NBUF, group, 0)
        for kk in range(CHUNKS_PER_TILE - (NBUF - 2), CHUNKS_PER_TILE):
            wait_scatter(kk % NBUF)
        plsc.subcore_barrier()

        for j in range(ROWS_PER_TILE // EDGE_CHUNK):
            r = row0 + j * EDGE_CHUNK
            pltpu.sync_copy(acc.at[pl.ds(r, EDGE_CHUNK)], zero_b)
            pltpu.sync_copy(zero_b, out_hbm.at[pl.ds(cid * ACC_ROWS + r, EDGE_CHUNK)])

    return k(dst2)


def _sc_scatter(u, idx2, zrows):
    mesh = plsc.VectorSubcoreMesh(core_axis_name="c", subcore_axis_name="s")

    @functools.partial(
        pl.kernel,
        mesh=mesh,
        out_type=jax.ShapeDtypeStruct((NC * ACC_ROWS, H), jnp.float32),
        compiler_params=pltpu.CompilerParams(use_tc_tiling_on_sc=False),
        scratch_types=[
            pltpu.VMEM_SHARED((ACC_ROWS, H), jnp.float32),
            pltpu.VMEM_SHARED((ACC_ROWS, H), jnp.float32),
            pltpu.VMEM((NBUF, 2, EDGE_CHUNK), jnp.int32),
        ]
        + [pltpu.VMEM((EDGE_CHUNK, H), jnp.float32)] * NBUF
        + [pltpu.SemaphoreType.DMA] * (3 * NBUF),
    )
    def k(u_hbm, idx_hbm, z_hbm, out_hbm, acc, u_s, idxI, *rest):
        rows = rest[:NBUF]
        isems = rest[NBUF:2 * NBUF]
        gsems = rest[2 * NBUF:3 * NBUF]
        ssems = rest[3 * NBUF:]
        cid = lax.axis_index("c")
        sid = lax.axis_index("s")
        wid = cid * NS + sid
        crow0 = wid * CHUNKS_PER_TILE

        row0 = sid * ROWS_PER_TILE
        tail = (N // 16) * 16 - (N // EDGE_CHUNK) * EDGE_CHUNK
        t0 = (N // EDGE_CHUNK) * EDGE_CHUNK
        for j in range(ROWS_PER_TILE // EDGE_CHUNK):
            r = row0 + j * EDGE_CHUNK
            pltpu.async_copy(z_hbm, acc.at[pl.ds(r, EDGE_CHUNK)], isems[0])

            @pl.when(r + EDGE_CHUNK <= N)
            def _():
                pltpu.async_copy(u_hbm.at[pl.ds(r, EDGE_CHUNK)],
                                 u_s.at[pl.ds(r, EDGE_CHUNK)], isems[1])

        @pl.when(sid == NS - 1)
        def _():
            pltpu.async_copy(u_hbm.at[pl.ds(t0, tail)],
                             u_s.at[pl.ds(t0, tail)], isems[1])

        for j in range(ROWS_PER_TILE // EDGE_CHUNK):
            r = row0 + j * EDGE_CHUNK
            pltpu.make_async_copy(z_hbm, acc.at[pl.ds(r, EDGE_CHUNK)],
                                  isems[0]).wait()

            @pl.when(r + EDGE_CHUNK <= N)
            def _():
                pltpu.make_async_copy(u_hbm.at[pl.ds(r, EDGE_CHUNK)],
                                      u_s.at[pl.ds(r, EDGE_CHUNK)],
                                      isems[1]).wait()

        @pl.when(sid == NS - 1)
        def _():
            pltpu.make_async_copy(u_hbm.at[pl.ds(t0, tail)],
                                  u_s.at[pl.ds(t0, tail)], isems[1]).wait()

        plsc.subcore_barrier()

        def fire_idx(kk, b):
            pltpu.async_copy(idx_hbm.at[crow0 + kk], idxI.at[b], isems[b])

        def wait_idx(b):
            pltpu.make_async_copy(idx_hbm.at[0], idxI.at[b], isems[b]).wait()

        def fire_gather(b):
            pltpu.async_copy(u_s.at[idxI.at[b, 0]], rows[b], gsems[b])

        def wait_gather(b):
            pltpu.make_async_copy(u_s.at[idxI.at[b, 0]], rows[b], gsems[b]).wait()

        def fire_scatter(b):
            pltpu.async_copy(rows[b], acc.at[idxI.at[b, 1]], ssems[b], add=True)

        def wait_scatter(b):
            pltpu.make_async_copy(rows[b], acc.at[idxI.at[b, 1]], ssems[b]).wait()

        fire_idx(0, 0)
        fire_idx(1, 1)
        wait_idx(0)
        fire_gather(0)

        CH = CHUNKS_PER_TILE

        def group(g, carry):
            for b in range(NBUF):
                kk = g * NBUF + b
                b1 = (b + 1) % NBUF
                b2 = (b + 2) % NBUF

                @pl.when(kk >= NBUF - 2)
                def _():
                    wait_scatter(b2)

                @pl.when(kk + 2 < CH)
                def _():
                    fire_idx(kk + 2, b2)

                @pl.when(kk + 1 < CH)
                def _():
                    wait_idx(b1)
                    fire_gather(b1)

                wait_gather(b)
                fire_scatter(b)
            return carry

        lax.fori_loop(0, CH // NBUF, group, 0)
        for kk in range(CH - (NBUF - 2), CH):
            wait_scatter(kk % NBUF)
        plsc.subcore_barrier()

        for j in range(ROWS_PER_TILE // EDGE_CHUNK):
            r = row0 + j * EDGE_CHUNK
            pltpu.async_copy(acc.at[pl.ds(r, EDGE_CHUNK)],
                             out_hbm.at[pl.ds(cid * ACC_ROWS + r, EDGE_CHUNK)],
                             isems[0])
        for j in range(ROWS_PER_TILE // EDGE_CHUNK):
            r = row0 + j * EDGE_CHUNK
            pltpu.make_async_copy(acc.at[pl.ds(r, EDGE_CHUNK)],
                                  out_hbm.at[pl.ds(cid * ACC_ROWS + r, EDGE_CHUNK)],
                                  isems[0]).wait()

    return k(u, idx2, zrows)


def _dinv_from_deg(deg_blk):
    deg = jnp.sum(deg_blk, axis=(0, 2)) * (1.0 / DEG_W) + 1.0
    return jax.lax.rsqrt(deg)


def _k1_body(x_ref, w1_ref, deg_ref, u1_ref):
    dinv = _dinv_from_deg(deg_ref[...])
    y = jnp.dot(x_ref[...], w1_ref[...], precision=_HI,
                preferred_element_type=jnp.float32)
    u1_ref[...] = y * dinv[:, None]


def _tc_k1(x, W1, deg3):
    return pl.pallas_call(
        _k1_body,
        grid=(GRID,),
        in_specs=[
            pl.BlockSpec((BN, F_IN), lambda i: (i, 0)),
            pl.BlockSpec((F_IN, H), lambda i: (0, 0)),
            pl.BlockSpec((NC, BN, DEG_W), lambda i: (0, i, 0)),
        ],
        out_specs=pl.BlockSpec((BN, H), lambda i: (i, 0)),
        out_shape=jax.ShapeDtypeStruct((N, H), jnp.float32),
    )(x, W1, deg3)


def _k2_body(s_ref, u1_ref, deg_ref, w2_ref, b1_ref, x1_ref, u2_ref):
    dinv = _dinv_from_deg(deg_ref[...])
    s = s_ref[...]
    x1 = jnp.maximum((s[0] + s[1] + u1_ref[...]) * dinv[:, None] + b1_ref[...], 0.0)
    x1_ref[...] = x1
    u2_ref[...] = jnp.dot(x1, w2_ref[...], precision=_HI,
                          preferred_element_type=jnp.float32) * dinv[:, None]


def _tc_k2(s1, u1, deg3, W2, b1r):
    return pl.pallas_call(
        _k2_body,
        grid=(GRID,),
        in_specs=[
            pl.BlockSpec((NC, BN, H), lambda i: (0, i, 0)),
            pl.BlockSpec((BN, H), lambda i: (i, 0)),
            pl.BlockSpec((NC, BN, DEG_W), lambda i: (0, i, 0)),
            pl.BlockSpec((H, H), lambda i: (0, 0)),
            pl.BlockSpec((1, H), lambda i: (0, 0)),
        ],
        out_specs=[
            pl.BlockSpec((BN, H), lambda i: (i, 0)),
            pl.BlockSpec((BN, H), lambda i: (i, 0)),
        ],
        out_shape=[
            jax.ShapeDtypeStruct((N, H), jnp.float32),
            jax.ShapeDtypeStruct((N, H), jnp.float32),
        ],
    )(s1, u1, deg3, W2, b1r)


def _lstm_step(xt, h_prev, wihT, whhT, bsum, c_prev):
    g = jnp.dot(xt, wihT, preferred_element_type=jnp.float32) + bsum
    if h_prev is not None:
        g = g + jnp.dot(h_prev, whhT, preferred_element_type=jnp.float32)
    i = jax.nn.sigmoid(g[:, 0:LSTM_H])
    f = jax.nn.sigmoid(g[:, LSTM_H:2 * LSTM_H])
    gg = jnp.tanh(g[:, 2 * LSTM_H:3 * LSTM_H])
    o = jax.nn.sigmoid(g[:, 3 * LSTM_H:4 * LSTM_H])
    c = i * gg if c_prev is None else f * c_prev + i * gg
    return o * jnp.tanh(c), c


def _k3_body(s_ref, u2_ref, x1_ref, deg_ref, b2_ref,
             wihf_ref, whhf_ref, bf_ref, wihr_ref, whhr_ref, br_ref,
             wa_ref, u3_ref):
    dinv = _dinv_from_deg(deg_ref[...])
    s = s_ref[...]
    x1 = x1_ref[...]
    x2 = jnp.maximum((s[0] + s[1] + u2_ref[...]) * dinv[:, None] + b2_ref[...], 0.0)

    bf = bf_ref[...]
    br = br_ref[...]
    h1, c1 = _lstm_step(x1, None, wihf_ref[...], None, bf, None)
    h2, _ = _lstm_step(x2, h1, wihf_ref[...], whhf_ref[...], bf, c1)
    ha, ca = _lstm_step(x2, None, wihr_ref[...], None, br, None)
    hb, _ = _lstm_step(x1, ha, wihr_ref[...], whhr_ref[...], br, ca)

    wa = wa_ref[...]
    sc0 = jnp.sum(h1 * wa[0][None, :], axis=1) + jnp.sum(hb * wa[1][None, :], axis=1)
    sc1 = jnp.sum(h2 * wa[0][None, :], axis=1) + jnp.sum(ha * wa[1][None, :], axis=1)
    m = jnp.maximum(sc0, sc1)
    e0 = jnp.exp(sc0 - m)
    e1 = jnp.exp(sc1 - m)
    inv = 1.0 / (e0 + e1)
    xjk = (e0 * inv)[:, None] * x1 + (e1 * inv)[:, None] * x2
    u3_ref[...] = xjk * dinv[:, None]


def _tc_k3(s2, u2, x1, deg3, b2r, wihf, whhf, bfr, wihr, whhr, brr, wa):
    return pl.pallas_call(
        _k3_body,
        grid=(GRID,),
        in_specs=[
            pl.BlockSpec((NC, BN, H), lambda i: (0, i, 0)),
            pl.BlockSpec((BN, H), lambda i: (i, 0)),
            pl.BlockSpec((BN, H), lambda i: (i, 0)),
            pl.BlockSpec((NC, BN, DEG_W), lambda i: (0, i, 0)),
            pl.BlockSpec((1, H), lambda i: (0, 0)),
            pl.BlockSpec((H, 4 * LSTM_H), lambda i: (0, 0)),
            pl.BlockSpec((LSTM_H, 4 * LSTM_H), lambda i: (0, 0)),
            pl.BlockSpec((1, 4 * LSTM_H), lambda i: (0, 0)),
            pl.BlockSpec((H, 4 * LSTM_H), lambda i: (0, 0)),
            pl.BlockSpec((LSTM_H, 4 * LSTM_H), lambda i: (0, 0)),
            pl.BlockSpec((1, 4 * LSTM_H), lambda i: (0, 0)),
            pl.BlockSpec((2, LSTM_H), lambda i: (0, 0)),
        ],
        out_specs=pl.BlockSpec((BN, H), lambda i: (i, 0)),
        out_shape=jax.ShapeDtypeStruct((N, H), jnp.float32),
    )(s2, u2, x1, deg3, b2r, wihf, whhf, bfr, wihr, whhr, brr, wa)


def _k4_body(s_ref, u3_ref, deg_ref, w3_ref, b3_ref, out_ref):
    dinv = _dinv_from_deg(deg_ref[...])
    s = s_ref[...]
    xp = (s[0] + s[1] + u3_ref[...]) * dinv[:, None]
    logits = jnp.dot(xp, w3_ref[...], precision=_HI,
                     preferred_element_type=jnp.float32) + b3_ref[...]
    m = jnp.max(logits, axis=1, keepdims=True)
    lse = jnp.log(jnp.sum(jnp.exp(logits - m), axis=1, keepdims=True)) + m
    out_ref[...] = logits - lse


def _tc_k4(s3, u3, deg3, W3, b3r):
    return pl.pallas_call(
        _k4_body,
        grid=(GRID,),
        in_specs=[
            pl.BlockSpec((NC, BN, H), lambda i: (0, i, 0)),
            pl.BlockSpec((BN, H), lambda i: (i, 0)),
            pl.BlockSpec((NC, BN, DEG_W), lambda i: (0, i, 0)),
            pl.BlockSpec((H, C_OUT), lambda i: (0, 0)),
            pl.BlockSpec((1, C_OUT), lambda i: (0, 0)),
        ],
        out_specs=pl.BlockSpec((BN, C_OUT), lambda i: (i, 0)),
        out_shape=jax.ShapeDtypeStruct((N, C_OUT), jnp.float32),
    )(s3, u3, deg3, W3, b3r)


def kernel(x, edge_index, W1, b1, W2, b2, Wih_f, Whh_f, bih_f, bhh_f,
           Wih_r, Whh_r, bih_r, bhh_r, Watt, batt, W3, b3):
    pad = EP - E
    srcp = jnp.concatenate([edge_index[0], jnp.zeros((pad,), jnp.int32)])
    dstp = jnp.concatenate([edge_index[1], jnp.full((pad,), N, jnp.int32)])
    src2 = srcp.reshape(EP // EDGE_CHUNK, EDGE_CHUNK)
    dst2 = dstp.reshape(EP // EDGE_CHUNK, EDGE_CHUNK)
    idx2 = jnp.stack([src2, dst2], axis=1)
    zrows = jnp.zeros((EDGE_CHUNK, H), jnp.float32)

    b1r = b1.reshape(1, H)
    b2r = b2.reshape(1, H)
    b3r = b3.reshape(1, C_OUT)
    wihf = Wih_f.T
    whhf = Whh_f.T
    wihr = Wih_r.T
    whhr = Whh_r.T
    bfr = (bih_f + bhh_f).reshape(1, 4 * LSTM_H)
    brr = (bih_r + bhh_r).reshape(1, 4 * LSTM_H)
    wa = Watt[:, 0].reshape(2, LSTM_H)

    degp = _sc_degree(dst2)
    deg3 = degp.reshape(NC, ACC_ROWS, DEG_W)

    u1 = _tc_k1(x, W1, deg3)
    s1 = _sc_scatter(u1, idx2, zrows).reshape(NC, ACC_ROWS, H)
    x1, u2 = _tc_k2(s1, u1, deg3, W2, b1r)

    s2 = _sc_scatter(u2, idx2, zrows).reshape(NC, ACC_ROWS, H)
    u3 = _tc_k3(s2, u2, x1, deg3, b2r, wihf, whhf, bfr, wihr, whhr, brr, wa)

    s3 = _sc_scatter(u3, idx2, zrows).reshape(NC, ACC_ROWS, H)
    return _tc_k4(s3, u3, deg3, W3, b3r)


# speedup vs baseline: 1.0430x; 1.0430x over previous
import functools

import jax
import jax.numpy as jnp
from jax import lax
from jax.experimental import pallas as pl
from jax.experimental.pallas import tpu as pltpu
from jax.experimental.pallas import tpu_sc as plsc

N = 10000
E = 320000
F_IN = 128
H = 64
C_OUT = 16
LSTM_H = 128

NC = 2
NS = 16
NW = NC * NS

ACC_ROWS = 10240
ROWS_PER_TILE = ACC_ROWS // NS
EDGE_CHUNK = 128
EP = 327680
EDGES_PER_TILE = EP // NW
CHUNKS_PER_TILE = EDGES_PER_TILE // EDGE_CHUNK
DEG_W = 16

BN = 2000
GRID = N // BN

_HI = jax.lax.Precision.DEFAULT


def _fill_buf(buf, nrow, ncolv, value):
    v = jnp.full((16,), value, jnp.float32)

    def body(t, carry):
        i = t // ncolv
        j = t % ncolv
        buf[i, pl.ds(j * 16, 16)] = v
        return carry

    lax.fori_loop(0, nrow * ncolv, body, 0)


NBUF = 5


def _sc_degree(dst2):
    mesh = plsc.VectorSubcoreMesh(core_axis_name="c", subcore_axis_name="s")

    @functools.partial(
        pl.kernel,
        mesh=mesh,
        out_type=jax.ShapeDtypeStruct((NC * ACC_ROWS, DEG_W), jnp.float32),
        compiler_params=pltpu.CompilerParams(use_tc_tiling_on_sc=False),
        scratch_types=[
            pltpu.VMEM_SHARED((ACC_ROWS, DEG_W), jnp.float32),
            pltpu.VMEM((NBUF, EDGE_CHUNK), jnp.int32),
            pltpu.VMEM((EDGE_CHUNK, DEG_W), jnp.float32),
            pltpu.VMEM((EDGE_CHUNK, DEG_W), jnp.float32),
        ]
        + [pltpu.SemaphoreType.DMA] * (2 * NBUF),
    )
    def k(dst_hbm, out_hbm, acc, dstI, ones_b, zero_b, *sems):
        cid = lax.axis_index("c")
        sid = lax.axis_index("s")
        wid = cid * NS + sid
        isems = sems[:NBUF]
        ssems = sems[NBUF:]

        _fill_buf(zero_b, EDGE_CHUNK, DEG_W // 16, 0.0)
        _fill_buf(ones_b, EDGE_CHUNK, DEG_W // 16, 1.0)

        row0 = sid * ROWS_PER_TILE
        for j in range(ROWS_PER_TILE // EDGE_CHUNK):
            pltpu.sync_copy(zero_b, acc.at[pl.ds(row0 + j * EDGE_CHUNK, EDGE_CHUNK)])
        plsc.subcore_barrier()

        crow0 = wid * CHUNKS_PER_TILE

        def fire_idx(kk, b):
            pltpu.async_copy(dst_hbm.at[crow0 + kk], dstI.at[b], isems[b])

        def wait_idx(b):
            pltpu.make_async_copy(dst_hbm.at[0], dstI.at[b], isems[b]).wait()

        def fire_scatter(b):
            pltpu.async_copy(ones_b, acc.at[dstI.at[b]], ssems[b], add=True)

        def wait_scatter(b):
            pltpu.make_async_copy(ones_b, acc.at[dstI.at[b]], ssems[b]).wait()

        fire_idx(0, 0)
        fire_idx(1, 1)

        def group(g, carry):
            for b in range(NBUF):
                kk = g * NBUF + b
                b2 = (b + 2) % NBUF

                @pl.when(kk >= NBUF - 2)
                def _():
                    wait_scatter(b2)

                @pl.when(kk + 2 < CHUNKS_PER_TILE)
                def _():
                    fire_idx(kk + 2, b2)

                wait_idx(b)
                fire_scatter(b)
            return carry

        lax.fori_loop(0, CHUNKS_PER_TILE // NBUF, group, 0)
        for kk in range(CHUNKS_PER_TILE - (NBUF - 2), CHUNKS_PER_TILE):
            wait_scatter(kk % NBUF)
        plsc.subcore_barrier()

        for j in range(ROWS_PER_TILE // EDGE_CHUNK):
            r = row0 + j * EDGE_CHUNK
            pltpu.sync_copy(acc.at[pl.ds(r, EDGE_CHUNK)], zero_b)
            pltpu.sync_copy(zero_b, out_hbm.at[pl.ds(cid * ACC_ROWS + r, EDGE_CHUNK)])

    return k(dst2)


def _sc_scatter(u, idx2, zrows):
    mesh = plsc.VectorSubcoreMesh(core_axis_name="c", subcore_axis_name="s")

    @functools.partial(
        pl.kernel,
        mesh=mesh,
        out_type=jax.ShapeDtypeStruct((NC * ACC_ROWS, H), jnp.float32),
        compiler_params=pltpu.CompilerParams(use_tc_tiling_on_sc=False),
        scratch_types=[
            pltpu.VMEM_SHARED((ACC_ROWS, H), jnp.float32),
            pltpu.VMEM_SHARED((ACC_ROWS, H), jnp.float32),
            pltpu.VMEM((NBUF, 2, EDGE_CHUNK), jnp.int32),
        ]
        + [pltpu.VMEM((EDGE_CHUNK, H), jnp.float32)] * NBUF
        + [pltpu.SemaphoreType.DMA] * (3 * NBUF),
    )
    def k(u_hbm, idx_hbm, z_hbm, out_hbm, acc, u_s, idxI, *rest):
        rows = rest[:NBUF]
        isems = rest[NBUF:2 * NBUF]
        gsems = rest[2 * NBUF:3 * NBUF]
        ssems = rest[3 * NBUF:]
        cid = lax.axis_index("c")
        sid = lax.axis_index("s")
        wid = cid * NS + sid
        crow0 = wid * CHUNKS_PER_TILE

        pltpu.sync_copy(z_hbm, rows[0])
        row0 = sid * ROWS_PER_TILE
        for j in range(ROWS_PER_TILE // EDGE_CHUNK):
            r = row0 + j * EDGE_CHUNK
            pltpu.sync_copy(rows[0], acc.at[pl.ds(r, EDGE_CHUNK)])

            @pl.when(r + EDGE_CHUNK <= N)
            def _():
                pltpu.sync_copy(u_hbm.at[pl.ds(r, EDGE_CHUNK)],
                                u_s.at[pl.ds(r, EDGE_CHUNK)])

        @pl.when(sid == NS - 1)
        def _():
            tail = (N // 16) * 16 - (N // EDGE_CHUNK) * EDGE_CHUNK
            t0 = (N // EDGE_CHUNK) * EDGE_CHUNK
            pltpu.sync_copy(u_hbm.at[pl.ds(t0, tail)], u_s.at[pl.ds(t0, tail)])

        plsc.subcore_barrier()

        def fire_idx(kk, b):
            pltpu.async_copy(idx_hbm.at[crow0 + kk], idxI.at[b], isems[b])

        def wait_idx(b):
            pltpu.make_async_copy(idx_hbm.at[0], idxI.at[b], isems[b]).wait()

        def fire_gather(b):
            pltpu.async_copy(u_s.at[idxI.at[b, 0]], rows[b], gsems[b])

        def wait_gather(b):
            pltpu.make_async_copy(u_s.at[idxI.at[b, 0]], rows[b], gsems[b]).wait()

        def fire_scatter(b):
            pltpu.async_copy(rows[b], acc.at[idxI.at[b, 1]], ssems[b], add=True)

        def wait_scatter(b):
            pltpu.make_async_copy(rows[b], acc.at[idxI.at[b, 1]], ssems[b]).wait()

        fire_idx(0, 0)
        fire_idx(1, 1)
        wait_idx(0)
        fire_gather(0)

        CH = CHUNKS_PER_TILE

        def group(g, carry):
            for b in range(NBUF):
                kk = g * NBUF + b
                b1 = (b + 1) % NBUF
                b2 = (b + 2) % NBUF

                @pl.when(kk >= NBUF - 2)
                def _():
                    wait_scatter(b2)

                @pl.when(kk + 2 < CH)
                def _():
                    fire_idx(kk + 2, b2)

                @pl.when(kk + 1 < CH)
                def _():
                    wait_idx(b1)
                    fire_gather(b1)

                wait_gather(b)
                fire_scatter(b)
            return carry

        lax.fori_loop(0, CH // NBUF, group, 0)
        for kk in range(CH - (NBUF - 2), CH):
            wait_scatter(kk % NBUF)
        plsc.subcore_barrier()

        for j in range(ROWS_PER_TILE // EDGE_CHUNK):
            r = row0 + j * EDGE_CHUNK
            pltpu.sync_copy(acc.at[pl.ds(r, EDGE_CHUNK)], rows[0])
            pltpu.sync_copy(rows[0], out_hbm.at[pl.ds(cid * ACC_ROWS + r, EDGE_CHUNK)])

    return k(u, idx2, zrows)


def _dinv_from_deg(deg_blk):
    deg = jnp.sum(deg_blk, axis=(0, 2)) * (1.0 / DEG_W) + 1.0
    return jax.lax.rsqrt(deg)


def _k1_body(x_ref, w1_ref, deg_ref, u1_ref):
    dinv = _dinv_from_deg(deg_ref[...])
    y = jnp.dot(x_ref[...], w1_ref[...], precision=_HI,
                preferred_element_type=jnp.float32)
    u1_ref[...] = y * dinv[:, None]


def _tc_k1(x, W1, deg3):
    return pl.pallas_call(
        _k1_body,
        grid=(GRID,),
        in_specs=[
            pl.BlockSpec((BN, F_IN), lambda i: (i, 0)),
            pl.BlockSpec((F_IN, H), lambda i: (0, 0)),
            pl.BlockSpec((NC, BN, DEG_W), lambda i: (0, i, 0)),
        ],
        out_specs=pl.BlockSpec((BN, H), lambda i: (i, 0)),
        out_shape=jax.ShapeDtypeStruct((N, H), jnp.float32),
    )(x, W1, deg3)


def _k2_body(s_ref, u1_ref, deg_ref, w2_ref, b1_ref, x1_ref, u2_ref):
    dinv = _dinv_from_deg(deg_ref[...])
    s = s_ref[...]
    x1 = jnp.maximum((s[0] + s[1] + u1_ref[...]) * dinv[:, None] + b1_ref[...], 0.0)
    x1_ref[...] = x1
    u2_ref[...] = jnp.dot(x1, w2_ref[...], precision=_HI,
                          preferred_element_type=jnp.float32) * dinv[:, None]


def _tc_k2(s1, u1, deg3, W2, b1r):
    return pl.pallas_call(
        _k2_body,
        grid=(GRID,),
        in_specs=[
            pl.BlockSpec((NC, BN, H), lambda i: (0, i, 0)),
            pl.BlockSpec((BN, H), lambda i: (i, 0)),
            pl.BlockSpec((NC, BN, DEG_W), lambda i: (0, i, 0)),
            pl.BlockSpec((H, H), lambda i: (0, 0)),
            pl.BlockSpec((1, H), lambda i: (0, 0)),
        ],
        out_specs=[
            pl.BlockSpec((BN, H), lambda i: (i, 0)),
            pl.BlockSpec((BN, H), lambda i: (i, 0)),
        ],
        out_shape=[
            jax.ShapeDtypeStruct((N, H), jnp.float32),
            jax.ShapeDtypeStruct((N, H), jnp.float32),
        ],
    )(s1, u1, deg3, W2, b1r)


def _lstm_step(xt, h_prev, wihT, whhT, bsum, c_prev):
    g = jnp.dot(xt, wihT, preferred_element_type=jnp.float32) + bsum
    if h_prev is not None:
        g = g + jnp.dot(h_prev, whhT, preferred_element_type=jnp.float32)
    i = jax.nn.sigmoid(g[:, 0:LSTM_H])
    f = jax.nn.sigmoid(g[:, LSTM_H:2 * LSTM_H])
    gg = jnp.tanh(g[:, 2 * LSTM_H:3 * LSTM_H])
    o = jax.nn.sigmoid(g[:, 3 * LSTM_H:4 * LSTM_H])
    c = i * gg if c_prev is None else f * c_prev + i * gg
    return o * jnp.tanh(c), c


def _k3_body(s_ref, u2_ref, x1_ref, deg_ref, b2_ref,
             wihf_ref, whhf_ref, bf_ref, wihr_ref, whhr_ref, br_ref,
             wa_ref, u3_ref):
    dinv = _dinv_from_deg(deg_ref[...])
    s = s_ref[...]
    x1 = x1_ref[...]
    x2 = jnp.maximum((s[0] + s[1] + u2_ref[...]) * dinv[:, None] + b2_ref[...], 0.0)

    bf = bf_ref[...]
    br = br_ref[...]
    h1, c1 = _lstm_step(x1, None, wihf_ref[...], None, bf, None)
    h2, _ = _lstm_step(x2, h1, wihf_ref[...], whhf_ref[...], bf, c1)
    ha, ca = _lstm_step(x2, None, wihr_ref[...], None, br, None)
    hb, _ = _lstm_step(x1, ha, wihr_ref[...], whhr_ref[...], br, ca)

    wa = wa_ref[...]
    sc0 = jnp.sum(h1 * wa[0][None, :], axis=1) + jnp.sum(hb * wa[1][None, :], axis=1)
    sc1 = jnp.sum(h2 * wa[0][None, :], axis=1) + jnp.sum(ha * wa[1][None, :], axis=1)
    m = jnp.maximum(sc0, sc1)
    e0 = jnp.exp(sc0 - m)
    e1 = jnp.exp(sc1 - m)
    inv = 1.0 / (e0 + e1)
    xjk = (e0 * inv)[:, None] * x1 + (e1 * inv)[:, None] * x2
    u3_ref[...] = xjk * dinv[:, None]


def _tc_k3(s2, u2, x1, deg3, b2r, wihf, whhf, bfr, wihr, whhr, brr, wa):
    return pl.pallas_call(
        _k3_body,
        grid=(GRID,),
        in_specs=[
            pl.BlockSpec((NC, BN, H), lambda i: (0, i, 0)),
            pl.BlockSpec((BN, H), lambda i: (i, 0)),
            pl.BlockSpec((BN, H), lambda i: (i, 0)),
            pl.BlockSpec((NC, BN, DEG_W), lambda i: (0, i, 0)),
            pl.BlockSpec((1, H), lambda i: (0, 0)),
            pl.BlockSpec((H, 4 * LSTM_H), lambda i: (0, 0)),
            pl.BlockSpec((LSTM_H, 4 * LSTM_H), lambda i: (0, 0)),
            pl.BlockSpec((1, 4 * LSTM_H), lambda i: (0, 0)),
            pl.BlockSpec((H, 4 * LSTM_H), lambda i: (0, 0)),
            pl.BlockSpec((LSTM_H, 4 * LSTM_H), lambda i: (0, 0)),
            pl.BlockSpec((1, 4 * LSTM_H), lambda i: (0, 0)),
            pl.BlockSpec((2, LSTM_H), lambda i: (0, 0)),
        ],
        out_specs=pl.BlockSpec((BN, H), lambda i: (i, 0)),
        out_shape=jax.ShapeDtypeStruct((N, H), jnp.float32),
    )(s2, u2, x1, deg3, b2r, wihf, whhf, bfr, wihr, whhr, brr, wa)


def _k4_body(s_ref, u3_ref, deg_ref, w3_ref, b3_ref, out_ref):
    dinv = _dinv_from_deg(deg_ref[...])
    s = s_ref[...]
    xp = (s[0] + s[1] + u3_ref[...]) * dinv[:, None]
    logits = jnp.dot(xp, w3_ref[...], precision=_HI,
                     preferred_element_type=jnp.float32) + b3_ref[...]
    m = jnp.max(logits, axis=1, keepdims=True)
    lse = jnp.log(jnp.sum(jnp.exp(logits - m), axis=1, keepdims=True)) + m
    out_ref[...] = logits - lse


def _tc_k4(s3, u3, deg3, W3, b3r):
    return pl.pallas_call(
        _k4_body,
        grid=(GRID,),
        in_specs=[
            pl.BlockSpec((NC, BN, H), lambda i: (0, i, 0)),
            pl.BlockSpec((BN, H), lambda i: (i, 0)),
            pl.BlockSpec((NC, BN, DEG_W), lambda i: (0, i, 0)),
            pl.BlockSpec((H, C_OUT), lambda i: (0, 0)),
            pl.BlockSpec((1, C_OUT), lambda i: (0, 0)),
        ],
        out_specs=pl.BlockSpec((BN, C_OUT), lambda i: (i, 0)),
        out_shape=jax.ShapeDtypeStruct((N, C_OUT), jnp.float32),
    )(s3, u3, deg3, W3, b3r)


def kernel(x, edge_index, W1, b1, W2, b2, Wih_f, Whh_f, bih_f, bhh_f,
           Wih_r, Whh_r, bih_r, bhh_r, Watt, batt, W3, b3):
    pad = EP - E
    srcp = jnp.concatenate([edge_index[0], jnp.zeros((pad,), jnp.int32)])
    dstp = jnp.concatenate([edge_index[1], jnp.full((pad,), N, jnp.int32)])
    src2 = srcp.reshape(EP // EDGE_CHUNK, EDGE_CHUNK)
    dst2 = dstp.reshape(EP // EDGE_CHUNK, EDGE_CHUNK)
    idx2 = jnp.stack([src2, dst2], axis=1)
    zrows = jnp.zeros((EDGE_CHUNK, H), jnp.float32)

    b1r = b1.reshape(1, H)
    b2r = b2.reshape(1, H)
    b3r = b3.reshape(1, C_OUT)
    wihf = Wih_f.T
    whhf = Whh_f.T
    wihr = Wih_r.T
    whhr = Whh_r.T
    bfr = (bih_f + bhh_f).reshape(1, 4 * LSTM_H)
    brr = (bih_r + bhh_r).reshape(1, 4 * LSTM_H)
    wa = Watt[:, 0].reshape(2, LSTM_H)

    degp = _sc_degree(dst2)
    deg3 = degp.reshape(NC, ACC_ROWS, DEG_W)

    u1 = _tc_k1(x, W1, deg3)
    s1 = _sc_scatter(u1, idx2, zrows).reshape(NC, ACC_ROWS, H)
    x1, u2 = _tc_k2(s1, u1, deg3, W2, b1r)

    s2 = _sc_scatter(u2, idx2, zrows).reshape(NC, ACC_ROWS, H)
    u3 = _tc_k3(s2, u2, x1, deg3, b2r, wihf, whhf, bfr, wihr, whhr, brr, wa)

    s3 = _sc_scatter(u3, idx2, zrows).reshape(NC, ACC_ROWS, H)
    return _tc_k4(s3, u3, deg3, W3, b3r)
